# Initial kernel scaffold; baseline (speedup 1.0000x reference)
import jax
import jax.numpy as jnp
from jax.experimental import pallas as pl


def kernel(card_indices, table):
    raise NotImplementedError("write your pallas kernel here")


# speedup vs baseline: 4.2756x; 4.2756x over previous
import functools

import jax
import jax.numpy as jnp
from jax import lax
from jax.experimental import pallas as pl
from jax.experimental.pallas import tpu as pltpu
from jax.experimental.pallas import tpu_sc as plsc

ROWS, FEAT = 52, 5
N = 16384 * 50
NW = 32
PER_W = N // NW
NCHUNK = 4
C = PER_W // NCHUNK
VPC = C // 16
TPAD = 320


def _body(idx_hbm, table_hbm, out_hbm, table_v, idx_v, out_v):
    wid = lax.axis_index("s") * 2 + lax.axis_index("c")
    base = wid * PER_W
    pltpu.sync_copy(table_hbm, table_v)
    pltpu.sync_copy(idx_hbm.at[pl.ds(base, PER_W)], idx_v)
    iota = lax.iota(jnp.int32, 16)
    pos = [iota * FEAT + f for f in range(FEAT)]
    for c in range(NCHUNK):
        def chunk_body(i, _, c=c):
            vi = idx_v[pl.ds(c * C + i * 16, 16)]
            tb = vi * FEAT
            off = i * (16 * FEAT)
            for f in range(FEAT):
                g = plsc.load_gather(table_v, [tb + f])
                plsc.store_scatter(out_v, [pos[f] + off], g)
            return 0
        lax.fori_loop(0, VPC, chunk_body, 0)
        pltpu.sync_copy(out_v, out_hbm.at[pl.ds((base + c * C) * FEAT, C * FEAT)])


_mesh = plsc.VectorSubcoreMesh(core_axis_name="c", subcore_axis_name="s")

_lookup = pl.kernel(
    _body,
    out_type=jax.ShapeDtypeStruct((N * FEAT,), jnp.float32),
    mesh=_mesh,
    scratch_types=[
        pltpu.VMEM((TPAD,), jnp.float32),
        pltpu.VMEM((PER_W,), jnp.int32),
        pltpu.VMEM((C * FEAT,), jnp.float32),
    ],
    compiler_params=pltpu.CompilerParams(needs_layout_passes=False),
)


@jax.jit
def kernel(card_indices, table):
    idx_flat = card_indices.reshape(-1).astype(jnp.int32)
    table_flat = jnp.pad(table.reshape(-1), (0, TPAD - ROWS * FEAT))
    out = _lookup(idx_flat, table_flat)
    return out.reshape(card_indices.shape[0], card_indices.shape[1], FEAT)


# speedup vs baseline: 42.7235x; 9.9923x over previous
import jax
import jax.numpy as jnp
from jax import lax
from jax.experimental import pallas as pl
from jax.experimental.pallas import tpu as pltpu
from jax.experimental.pallas import tpu_sc as plsc

ROWS, FEAT = 52, 5
NI, NJ = 16384, 50
NW = 32
W = 256
SLABS = NI // (NW * W)
TROWS = 64


def _body(idx_hbm, tcols_hbm, out_hbm, table_v, idx_v, out_v):
    wid = lax.axis_index("s") * 2 + lax.axis_index("c")
    pltpu.sync_copy(tcols_hbm, table_v)
    iota16 = lax.iota(jnp.int32, 16)
    for s in range(SLABS):
        i0 = (wid * SLABS + s) * W
        pltpu.sync_copy(idx_hbm.at[:, pl.ds(i0, W)], idx_v)

        def j_body(j, _):
            for b in range(W // 16):
                vi = idx_v[j, pl.ds(b * 16, 16)]
                for f in range(FEAT):
                    g = plsc.load_gather(table_v, [vi + f * TROWS])
                    out_v[f, j, pl.ds(b * 16, 16)] = g
            return 0

        lax.fori_loop(0, NJ, j_body, 0)
        pltpu.sync_copy(out_v, out_hbm.at[:, :, pl.ds(i0, W)])


_mesh = plsc.VectorSubcoreMesh(core_axis_name="c", subcore_axis_name="s")

_lookup = pl.kernel(
    _body,
    out_type=jax.ShapeDtypeStruct((FEAT, NJ, NI), jnp.float32),
    mesh=_mesh,
    scratch_types=[
        pltpu.VMEM((FEAT * TROWS,), jnp.float32),
        pltpu.VMEM((NJ, W), jnp.int32),
        pltpu.VMEM((FEAT, NJ, W), jnp.float32),
    ],
    compiler_params=pltpu.CompilerParams(needs_layout_passes=False),
)


@jax.jit
def kernel(card_indices, table):
    idx_t = card_indices.T
    tcols = jnp.pad(table.T, ((0, 0), (0, TROWS - ROWS)))
    a = _lookup(idx_t, tcols.reshape(-1))
    return jnp.transpose(a, (2, 1, 0))


# speedup vs baseline: 47.5028x; 1.1119x over previous
import jax
import jax.numpy as jnp
from jax import lax
from jax.experimental import pallas as pl
from jax.experimental.pallas import tpu as pltpu
from jax.experimental.pallas import tpu_sc as plsc

ROWS, FEAT = 52, 5
NI, NJ = 16384, 50
NW = 32
W = 256
SLABS = NI // (NW * W)
TROWS = 64
HALVES = ((0, 24), (24, 26))


def _body(idx_hbm, tcols_hbm, out_hbm,
          table_v, idx_v0, idx_v1, out_v,
          in_sem0, in_sem1, out_semA, out_semB):
    idx_bufs = (idx_v0, idx_v1)
    out_sems = (out_semA, out_semB)
    wid = lax.axis_index("s") * 2 + lax.axis_index("c")
    base = wid * SLABS * W
    in_copies = [
        pltpu.async_copy(idx_hbm.at[:, pl.ds(base + s * W, W)],
                         idx_bufs[s], (in_sem0, in_sem1)[s])
        for s in range(SLABS)
    ]
    pltpu.sync_copy(tcols_hbm, table_v)
    pending = {}
    for s in range(SLABS):
        idx_v = idx_bufs[s]
        in_copies[s].wait()
        for h, (j0, nrows) in enumerate(HALVES):
            if j0 in pending:
                pending[j0].wait()

            def j_body(j, _, idx_v=idx_v):
                for b in range(W // 16):
                    vi = idx_v[j, pl.ds(b * 16, 16)]
                    for f in range(FEAT):
                        g = plsc.load_gather(table_v, [vi + f * TROWS])
                        out_v[f, j, pl.ds(b * 16, 16)] = g
                return 0

            lax.fori_loop(j0, j0 + nrows, j_body, 0)
            pending[j0] = pltpu.async_copy(
                out_v.at[:, pl.ds(j0, nrows), :],
                out_hbm.at[:, pl.ds(j0, nrows), pl.ds(base + s * W, W)],
                out_sems[h])
    for d in pending.values():
        d.wait()


_mesh = plsc.VectorSubcoreMesh(core_axis_name="c", subcore_axis_name="s")

_lookup = pl.kernel(
    _body,
    out_type=jax.ShapeDtypeStruct((FEAT, NJ, NI), jnp.float32),
    mesh=_mesh,
    scratch_types=[
        pltpu.VMEM((FEAT * TROWS,), jnp.float32),
        pltpu.VMEM((NJ, W), jnp.int32),
        pltpu.VMEM((NJ, W), jnp.int32),
        pltpu.VMEM((FEAT, NJ, W), jnp.float32),
        pltpu.SemaphoreType.DMA,
        pltpu.SemaphoreType.DMA,
        pltpu.SemaphoreType.DMA,
        pltpu.SemaphoreType.DMA,
    ],
    compiler_params=pltpu.CompilerParams(needs_layout_passes=False),
)


@jax.jit
def kernel(card_indices, table):
    idx_t = card_indices.T
    tcols = jnp.pad(table.T, ((0, 0), (0, TROWS - ROWS)))
    a = _lookup(idx_t, tcols.reshape(-1))
    return jnp.transpose(a, (2, 1, 0))


# speedup vs baseline: 67.2892x; 1.4165x over previous
import jax
import jax.numpy as jnp
from jax import lax
from jax.experimental import pallas as pl
from jax.experimental.pallas import tpu as pltpu
from jax.experimental.pallas import tpu_sc as plsc

ROWS, FEAT = 52, 5
NI, NJ = 16384, 50
NW = 32
W = 256
SLABS = NI // (NW * W)
TROWS = 64
HALVES = ((0, 24), (24, 26))


def _body(idx_hbm, tcols_hbm, out_hbm,
          table_v, idx_v0, idx_v1, out_v,
          in_sem0, in_sem1, out_semA, out_semB):
    idx_bufs = (idx_v0, idx_v1)
    out_sems = (out_semA, out_semB)
    wid = lax.axis_index("s") * 2 + lax.axis_index("c")
    base = wid * SLABS * W
    in_copies = [
        pltpu.async_copy(idx_hbm.at[:, pl.ds(base + s * W, W)],
                         idx_bufs[s], (in_sem0, in_sem1)[s])
        for s in range(SLABS)
    ]
    pltpu.sync_copy(tcols_hbm, table_v)
    pending = {}
    for s in range(SLABS):
        idx_v = idx_bufs[s]
        in_copies[s].wait()
        for h, (j0, nrows) in enumerate(HALVES):
            if j0 in pending:
                pending[j0].wait()

            @plsc.parallel_loop(j0, j0 + nrows, unroll=2)
            def j_body(j, idx_v=idx_v):
                for b in range(W // 16):
                    vi = idx_v[j, pl.ds(b * 16, 16)]
                    for f in range(FEAT):
                        g = plsc.load_gather(table_v, [vi + f * TROWS])
                        out_v[f, j, pl.ds(b * 16, 16)] = g
            pending[j0] = pltpu.async_copy(
                out_v.at[:, pl.ds(j0, nrows), :],
                out_hbm.at[:, pl.ds(j0, nrows), pl.ds(base + s * W, W)],
                out_sems[h])
    for d in pending.values():
        d.wait()


_mesh = plsc.VectorSubcoreMesh(core_axis_name="c", subcore_axis_name="s")

_lookup = pl.kernel(
    _body,
    out_type=jax.ShapeDtypeStruct((FEAT, NJ, NI), jnp.float32),
    mesh=_mesh,
    scratch_types=[
        pltpu.VMEM((FEAT * TROWS,), jnp.float32),
        pltpu.VMEM((NJ, W), jnp.int32),
        pltpu.VMEM((NJ, W), jnp.int32),
        pltpu.VMEM((FEAT, NJ, W), jnp.float32),
        pltpu.SemaphoreType.DMA,
        pltpu.SemaphoreType.DMA,
        pltpu.SemaphoreType.DMA,
        pltpu.SemaphoreType.DMA,
    ],
    compiler_params=pltpu.CompilerParams(needs_layout_passes=False),
)


@jax.jit
def kernel(card_indices, table):
    idx_t = card_indices.T
    tcols = jnp.pad(table.T, ((0, 0), (0, TROWS - ROWS)))
    a = _lookup(idx_t, tcols.reshape(-1))
    return jnp.transpose(a, (2, 1, 0))


# speedup vs baseline: 67.4013x; 1.0017x over previous
import jax
import jax.numpy as jnp
from jax import lax
from jax.experimental import pallas as pl
from jax.experimental.pallas import tpu as pltpu
from jax.experimental.pallas import tpu_sc as plsc

ROWS, FEAT = 52, 5
NI, NJ = 16384, 50
NW = 32
W = 256
SLABS = NI // (NW * W)
TROWS = 64
HALVES = ((0, 24), (24, 26))


def _body(idx_hbm, tcols_hbm, out_hbm,
          table_v, idx_v0, idx_v1, out_v,
          in_sem0, in_sem1, out_semA, out_semB):
    idx_bufs = (idx_v0, idx_v1)
    out_sems = (out_semA, out_semB)
    wid = lax.axis_index("c") * 16 + lax.axis_index("s")
    base = wid * SLABS * W
    in_copies = [
        pltpu.async_copy(idx_hbm.at[:, pl.ds(base + s * W, W)],
                         idx_bufs[s], (in_sem0, in_sem1)[s])
        for s in range(SLABS)
    ]
    pltpu.sync_copy(tcols_hbm, table_v)
    pending = {}
    for s in range(SLABS):
        idx_v = idx_bufs[s]
        in_copies[s].wait()
        for h, (j0, nrows) in enumerate(HALVES):
            if j0 in pending:
                pending[j0].wait()

            @plsc.parallel_loop(j0, j0 + nrows, unroll=2)
            def j_body(j, idx_v=idx_v):
                for b in range(W // 16):
                    vi = idx_v[j, pl.ds(b * 16, 16)]
                    for f in range(FEAT):
                        g = plsc.load_gather(table_v, [vi + f * TROWS])
                        out_v[f, j, pl.ds(b * 16, 16)] = g
            pending[j0] = pltpu.async_copy(
                out_v.at[:, pl.ds(j0, nrows), :],
                out_hbm.at[:, pl.ds(j0, nrows), pl.ds(base + s * W, W)],
                out_sems[h])
    for d in pending.values():
        d.wait()


_mesh = plsc.VectorSubcoreMesh(core_axis_name="c", subcore_axis_name="s")

_lookup = pl.kernel(
    _body,
    out_type=jax.ShapeDtypeStruct((FEAT, NJ, NI), jnp.float32),
    mesh=_mesh,
    scratch_types=[
        pltpu.VMEM((FEAT * TROWS,), jnp.float32),
        pltpu.VMEM((NJ, W), jnp.int32),
        pltpu.VMEM((NJ, W), jnp.int32),
        pltpu.VMEM((FEAT, NJ, W), jnp.float32),
        pltpu.SemaphoreType.DMA,
        pltpu.SemaphoreType.DMA,
        pltpu.SemaphoreType.DMA,
        pltpu.SemaphoreType.DMA,
    ],
    compiler_params=pltpu.CompilerParams(needs_layout_passes=False),
)


@jax.jit
def kernel(card_indices, table):
    idx_t = card_indices.T
    tcols = jnp.pad(table.T, ((0, 0), (0, TROWS - ROWS)))
    a = _lookup(idx_t, tcols.reshape(-1))
    return jnp.transpose(a, (2, 1, 0))


# speedup vs baseline: 92.8654x; 1.3778x over previous
import jax
import jax.numpy as jnp
from jax import lax
from jax.experimental import pallas as pl
from jax.experimental.pallas import tpu as pltpu
from jax.experimental.pallas import tpu_sc as plsc

ROWS, FEAT = 52, 5
NI, NJ = 16384, 50
NW = 32
W = 256
SLABS = NI // (NW * W)
TROWS = 64
HALVES = ((0, 24), (24, 26))


def _body(idx_hbm, tcols_hbm, out_hbm,
          table_v, idx_v0, idx_v1, out_v,
          in_sem0, in_sem1, out_semA, out_semB):
    idx_bufs = (idx_v0, idx_v1)
    out_sems = (out_semA, out_semB)
    wid = lax.axis_index("c") * 16 + lax.axis_index("s")
    base = wid * SLABS * W
    in_copies = [
        pltpu.async_copy(idx_hbm.at[:, pl.ds(base + s * W, W)],
                         idx_bufs[s], (in_sem0, in_sem1)[s])
        for s in range(SLABS)
    ]
    pltpu.sync_copy(tcols_hbm, table_v)
    pending = {}
    for s in range(SLABS):
        idx_v = idx_bufs[s]
        in_copies[s].wait()
        for h, (j0, nrows) in enumerate(HALVES):
            if j0 in pending:
                pending[j0].wait()

            @plsc.parallel_loop(j0 * (W // 16), (j0 + nrows) * (W // 16),
                                unroll=4)
            def t_body(t, idx_v=idx_v):
                j = t >> 4
                o = (t & 15) << 4
                vi = idx_v[j, pl.ds(o, 16)]
                for f in range(FEAT):
                    g = plsc.load_gather(table_v, [vi + f * TROWS])
                    out_v[f, j, pl.ds(o, 16)] = g
            pending[j0] = pltpu.async_copy(
                out_v.at[:, pl.ds(j0, nrows), :],
                out_hbm.at[:, pl.ds(j0, nrows), pl.ds(base + s * W, W)],
                out_sems[h])
    for d in pending.values():
        d.wait()


_mesh = plsc.VectorSubcoreMesh(core_axis_name="c", subcore_axis_name="s")

_lookup = pl.kernel(
    _body,
    out_type=jax.ShapeDtypeStruct((FEAT, NJ, NI), jnp.float32),
    mesh=_mesh,
    scratch_types=[
        pltpu.VMEM((FEAT * TROWS,), jnp.float32),
        pltpu.VMEM((NJ, W), jnp.int32),
        pltpu.VMEM((NJ, W), jnp.int32),
        pltpu.VMEM((FEAT, NJ, W), jnp.float32),
        pltpu.SemaphoreType.DMA,
        pltpu.SemaphoreType.DMA,
        pltpu.SemaphoreType.DMA,
        pltpu.SemaphoreType.DMA,
    ],
    compiler_params=pltpu.CompilerParams(needs_layout_passes=False),
)


@jax.jit
def kernel(card_indices, table):
    idx_t = card_indices.T
    tcols = jnp.pad(table.T, ((0, 0), (0, TROWS - ROWS)))
    a = _lookup(idx_t, tcols.reshape(-1))
    return jnp.transpose(a, (2, 1, 0))


# speedup vs baseline: 94.2775x; 1.0152x over previous
import jax
import jax.numpy as jnp
from jax import lax
from jax.experimental import pallas as pl
from jax.experimental.pallas import tpu as pltpu
from jax.experimental.pallas import tpu_sc as plsc

ROWS, FEAT = 52, 5
NI, NJ = 16384, 50
NW = 32
W = 256
SLABS = NI // (NW * W)
TROWS = ROWS
HALVES = ((0, 24), (24, 26))


def _body(idx_hbm, tcols_hbm, out_hbm,
          table_v, idx_v0, idx_v1, out_v,
          in_sem0, in_sem1, out_semA, out_semB):
    idx_bufs = (idx_v0, idx_v1)
    out_sems = (out_semA, out_semB)
    wid = lax.axis_index("c") * 16 + lax.axis_index("s")
    base = wid * SLABS * W
    in_copies = [
        pltpu.async_copy(idx_hbm.at[:, pl.ds(base + s * W, W)],
                         idx_bufs[s], (in_sem0, in_sem1)[s])
        for s in range(SLABS)
    ]
    pltpu.sync_copy(tcols_hbm, table_v)
    pending = {}
    for s in range(SLABS):
        idx_v = idx_bufs[s]
        in_copies[s].wait()
        for h, (j0, nrows) in enumerate(HALVES):
            if j0 in pending:
                pending[j0].wait()

            @plsc.parallel_loop(j0 * (W // 16), (j0 + nrows) * (W // 16),
                                unroll=4)
            def t_body(t, idx_v=idx_v):
                j = t >> 4
                o = (t & 15) << 4
                vi = idx_v[j, pl.ds(o, 16)]
                for f in range(FEAT):
                    g = plsc.load_gather(table_v, [vi + f * TROWS])
                    out_v[f, j, pl.ds(o, 16)] = g
            pending[j0] = pltpu.async_copy(
                out_v.at[:, pl.ds(j0, nrows), :],
                out_hbm.at[:, pl.ds(j0, nrows), pl.ds(base + s * W, W)],
                out_sems[h])
    for d in pending.values():
        d.wait()


_mesh = plsc.VectorSubcoreMesh(core_axis_name="c", subcore_axis_name="s")

_lookup = pl.kernel(
    _body,
    out_type=jax.ShapeDtypeStruct((FEAT, NJ, NI), jnp.float32),
    mesh=_mesh,
    scratch_types=[
        pltpu.VMEM((FEAT * TROWS,), jnp.float32),
        pltpu.VMEM((NJ, W), jnp.int32),
        pltpu.VMEM((NJ, W), jnp.int32),
        pltpu.VMEM((FEAT, NJ, W), jnp.float32),
        pltpu.SemaphoreType.DMA,
        pltpu.SemaphoreType.DMA,
        pltpu.SemaphoreType.DMA,
        pltpu.SemaphoreType.DMA,
    ],
    compiler_params=pltpu.CompilerParams(needs_layout_passes=False),
)


@jax.jit
def kernel(card_indices, table):
    idx_t = card_indices.T
    tcols = table.T.reshape(-1)
    a = _lookup(idx_t, tcols)
    return jnp.transpose(a, (2, 1, 0))
